# Initial kernel scaffold; baseline (speedup 1.0000x reference)
import jax
import jax.numpy as jnp
from jax.experimental import pallas as pl


def kernel(coords, mask, residue_pdb_idx):
    raise NotImplementedError("write your pallas kernel here")


# speedup vs baseline: 106.1085x; 106.1085x over previous
import functools

import jax
import jax.numpy as jnp
import numpy as np
from jax.experimental import pallas as pl


def _build_d() -> np.ndarray:
    d = np.zeros((32, 63), dtype=np.float32)
    for t in range(3):
        for j in range(10):
            col = 21 * t + 10 + j
            d[10 * t + j, col] = -1.0
            if j == 0:
                d[30, col] = 1.0
            else:
                d[10 * t + (j - 1), col] = 1.0
    return d


_D = _build_d()


def _angles_kernel(q_ref, idx_ref, thr_ref, d_ref, out_ref):
    x = q_ref[0]
    idx = idx_ref[0]
    n = x.shape[1]

    xs = jnp.roll(x, -1, axis=1)
    idx_s = jnp.roll(idx, -1, axis=1)

    lane = jax.lax.broadcasted_iota(jnp.int32, (1, n), 1)
    good = jnp.logical_and(idx_s - idx == 1, lane < n - 1)

    N, CA, C = x[0:3], x[3:6], x[6:9]
    Nn, CAn = xs[0:3], xs[3:6]

    def cosine(v1, v2):
        dot = jnp.sum(v1 * v2, axis=0, keepdims=True)
        n1 = jnp.sqrt(jnp.sum(v1 * v1, axis=0, keepdims=True))
        n2 = jnp.sqrt(jnp.sum(v2 * v2, axis=0, keepdims=True))
        c = dot / (n1 * n2 + 1e-10)
        return jnp.clip(c, -1.0 + 1e-7, 1.0 - 1e-7)

    c1 = cosine(N - CA, C - CA)
    c2 = jnp.where(good, cosine(CA - C, Nn - C), 2.0)
    c3 = jnp.where(good, cosine(C - Nn, CAn - Nn), 2.0)

    row = jax.lax.broadcasted_iota(jnp.int32, (32, n), 0)
    cb = jnp.where(row < 10, jnp.broadcast_to(c1, (32, n)),
                   jnp.where(row < 20, jnp.broadcast_to(c2, (32, n)),
                             jnp.broadcast_to(c3, (32, n))))
    thr = thr_ref[:, 0:1]
    u_t = jnp.where(cb < thr, 1.0, 0.0).astype(jnp.float32)

    feats = jax.lax.dot_general(
        u_t, d_ref[...],
        dimension_numbers=(((0,), (0,)), ((), ())),
        preferred_element_type=jnp.float32)
    out_ref[0] = feats


@jax.jit
def kernel(coords, mask, residue_pdb_idx):
    del mask
    b, n = coords.shape[0], coords.shape[1]
    q = coords[:, :, 0:3, :].reshape(b, n, 9).transpose(0, 2, 1)
    idx3 = residue_pdb_idx.astype(jnp.int32).reshape(b, 1, n)

    limits = jnp.linspace(-jnp.pi, jnp.pi, 20)
    thr10 = jnp.cos(limits[10:])
    thr32 = jnp.concatenate(
        [jnp.tile(thr10, 3), jnp.array([4.0, -4.0], dtype=jnp.float32)])
    thr = jnp.broadcast_to(thr32[:, None], (32, 128))
    d = jnp.asarray(_D)

    out = pl.pallas_call(
        _angles_kernel,
        grid=(b,),
        in_specs=[
            pl.BlockSpec((1, 9, n), lambda i: (i, 0, 0)),
            pl.BlockSpec((1, 1, n), lambda i: (i, 0, 0)),
            pl.BlockSpec((32, 128), lambda i: (0, 0)),
            pl.BlockSpec((32, 63), lambda i: (0, 0)),
        ],
        out_specs=pl.BlockSpec((1, n, 63), lambda i: (i, 0, 0)),
        out_shape=jax.ShapeDtypeStruct((b, n, 63), jnp.float32),
    )(q, idx3, thr, d)
    return out
